# baseline (speedup 1.0000x reference)
import functools

import jax
import jax.numpy as jnp
from jax import lax
from jax.experimental import pallas as pl
from jax.experimental.pallas import tpu as pltpu
from jax.experimental.pallas import tpu_sc as plsc

LANES = 16


@functools.lru_cache(maxsize=None)
def _build_emb(B: int, D: int, V: int):
    info = plsc.get_sparse_core_info()
    NC, NS = info.num_cores, info.num_subcores
    NW = NC * NS
    assert B % NW == 0
    bpw = B // NW
    C = 32
    while bpw % (2 * C):
        C //= 2
    nouter = bpw // (2 * C)
    mesh = plsc.VectorSubcoreMesh(core_axis_name="c", subcore_axis_name="s")

    @functools.partial(
        pl.kernel,
        mesh=mesh,
        out_type=jax.ShapeDtypeStruct((B, D), jnp.float32),
        scratch_types=[
            pltpu.VMEM((bpw,), jnp.int32),
            pltpu.VMEM((V, D), jnp.float32),
            pltpu.VMEM((C, D), jnp.float32),
            pltpu.VMEM((C, D), jnp.float32),
            pltpu.SemaphoreType.DMA,
            pltpu.SemaphoreType.DMA,
        ],
    )
    def emb(idx_hbm, table_hbm, out_hbm, idx_v, tab_v, rows0, rows1, s0, s1):
        wid = lax.axis_index("s") * NC + lax.axis_index("c")
        base = wid * bpw
        pltpu.sync_copy(table_hbm, tab_v)
        pltpu.sync_copy(idx_hbm.at[pl.ds(base, bpw)], idx_v)

        def fill(i, buf):
            def grp_body(g, carry):
                v16 = idx_v[pl.ds(i * C + g * LANES, LANES)]
                for k in range(LANES):
                    row = v16[k]
                    r = g * LANES + k
                    for j in range(D // LANES):
                        buf[r, pl.ds(LANES * j, LANES)] = tab_v[
                            row, pl.ds(LANES * j, LANES)
                        ]
                return carry

            lax.fori_loop(0, C // LANES, grp_body, 0)

        def s_start(i, buf, sem):
            pltpu.async_copy(buf, out_hbm.at[pl.ds(base + i * C, C)], sem)

        def s_wait(buf, sem):
            pltpu.make_async_copy(buf, out_hbm.at[pl.ds(base, C)], sem).wait()

        def body(o, carry):
            i0 = 2 * o

            @pl.when(o > 0)
            def _():
                s_wait(rows0, s0)

            fill(i0, rows0)
            s_start(i0, rows0, s0)

            @pl.when(o > 0)
            def _():
                s_wait(rows1, s1)

            fill(i0 + 1, rows1)
            s_start(i0 + 1, rows1, s1)
            return carry

        lax.fori_loop(0, nouter, body, 0)
        s_wait(rows0, s0)
        s_wait(rows1, s1)

    return emb


def kernel(seqs, W_embed):
    batch, seq = seqs.shape
    V, D = W_embed.shape
    idx = seqs.reshape(-1).astype(jnp.int32)
    emb = _build_emb(batch * seq, D, V)
    out = emb(idx, W_embed)
    return out.reshape(batch, seq, D)


# speedup vs baseline: 6.2057x; 6.2057x over previous
import functools

import jax
import jax.numpy as jnp
from jax import lax
from jax.experimental import pallas as pl
from jax.experimental.pallas import tpu as pltpu
from jax.experimental.pallas import tpu_sc as plsc

LANES = 16


@functools.lru_cache(maxsize=None)
def _build_emb(B: int, D: int, V: int):
    info = plsc.get_sparse_core_info()
    NC, NS = info.num_cores, info.num_subcores
    NW = NC * NS
    assert B % (NW * LANES) == 0
    bpw = B // NW
    ngrp = bpw // LANES
    mesh = plsc.VectorSubcoreMesh(core_axis_name="c", subcore_axis_name="s")

    @functools.partial(
        pl.kernel,
        mesh=mesh,
        out_type=jax.ShapeDtypeStruct((B, D), jnp.float32),
        scratch_types=[
            pltpu.VMEM((bpw,), jnp.int32),
            pltpu.VMEM((V, D), jnp.float32),
            pltpu.SemaphoreType.DMA,
        ],
    )
    def emb(idx_hbm, table_hbm, out_hbm, idx_v, tab_v, sem):
        wid = lax.axis_index("s") * NC + lax.axis_index("c")
        base = wid * bpw
        pltpu.sync_copy(table_hbm, tab_v)
        pltpu.sync_copy(idx_hbm.at[pl.ds(base, bpw)], idx_v)

        def grp_body(g, carry):
            v16 = idx_v[pl.ds(g * LANES, LANES)]
            for k in range(LANES):
                row = v16[k]
                pltpu.async_copy(
                    tab_v.at[row], out_hbm.at[base + g * LANES + k], sem
                )
            return carry

        lax.fori_loop(0, ngrp, grp_body, 0)

        def drain(q, carry):
            pltpu.make_async_copy(
                tab_v.at[pl.ds(0, LANES)],
                out_hbm.at[pl.ds(base, LANES)],
                sem,
            ).wait()
            return carry

        lax.fori_loop(0, ngrp, drain, 0)

    return emb


def kernel(seqs, W_embed):
    batch, seq = seqs.shape
    V, D = W_embed.shape
    idx = seqs.reshape(-1).astype(jnp.int32)
    emb = _build_emb(batch * seq, D, V)
    out = emb(idx, W_embed)
    return out.reshape(batch, seq, D)
